# baseline (speedup 1.0000x reference)
import functools

import jax
import jax.numpy as jnp
from jax import lax
from jax.experimental import pallas as pl
from jax.experimental.pallas import tpu as pltpu
from jax.experimental.pallas import tpu_sc as plsc

N, D_IN, D_OUT, E = 4096, 2048, 2048, 8
P = 2 * N
BT = 128
P_PAD = P + E * BT
G = P_PAD // BT
NC, NS = 2, 16
NW = NC * NS
BLOCK_N = 1024


def _gate_kernel(x_ref, gw_ref, gb_ref, w8_ref):
    gs = lax.dot_general(
        x_ref[...], gw_ref[...], (((1,), (1,)), ((), ())),
        preferred_element_type=jnp.float32,
    ) + gb_ref[...]
    lane = lax.broadcasted_iota(jnp.int32, gs.shape, 1)
    m1 = jnp.max(gs, axis=1, keepdims=True)
    i1 = jnp.min(jnp.where(gs == m1, lane, E), axis=1, keepdims=True)
    masked = jnp.where(lane == i1, -jnp.inf, gs)
    m2 = jnp.max(masked, axis=1, keepdims=True)
    i2 = jnp.min(jnp.where(masked == m2, lane, E), axis=1, keepdims=True)
    w1 = 1.0 / (1.0 + jnp.exp(m2 - m1))
    w8_ref[...] = jnp.where(lane == i1, w1, jnp.where(lane == i2, 1.0 - w1, 0.0))


def _gate(x, gate_W, gate_b):
    return pl.pallas_call(
        _gate_kernel,
        grid=(N // BLOCK_N,),
        in_specs=[
            pl.BlockSpec((BLOCK_N, D_IN), lambda n: (n, 0)),
            pl.BlockSpec((E, D_IN), lambda n: (0, 0)),
            pl.BlockSpec((1, E), lambda n: (0, 0)),
        ],
        out_specs=pl.BlockSpec((BLOCK_N, E), lambda n: (n, 0)),
        out_shape=jax.ShapeDtypeStruct((N, E), jnp.float32),
    )(x, gate_W, gate_b.reshape(1, E))


@functools.lru_cache(maxsize=None)
def _make_sc_row_gather(n_rows, d, ch):
    per_w = n_rows // NW
    mesh = plsc.VectorSubcoreMesh(core_axis_name="c", subcore_axis_name="s")

    @functools.partial(
        pl.kernel,
        mesh=mesh,
        out_type=jax.ShapeDtypeStruct((n_rows, d), jnp.float32),
        scratch_types=[
            pltpu.VMEM((ch,), jnp.int32),
            pltpu.VMEM((ch, d), jnp.float32),
            pltpu.SemaphoreType.DMA,
        ],
    )
    def gather(table_hbm, idx_hbm, out_hbm, idx_v, rows_v, sem):
        wid = lax.axis_index("s") * NC + lax.axis_index("c")
        base = wid * per_w

        def body(i, carry):
            off = base + i * ch
            pltpu.sync_copy(idx_hbm.at[pl.ds(off, ch)], idx_v)
            pltpu.async_copy(table_hbm.at[idx_v], rows_v, sem).wait()
            pltpu.sync_copy(rows_v, out_hbm.at[pl.ds(off, ch)])
            return carry

        lax.fori_loop(0, per_w // ch, body, 0)

    return gather


def _sc_row_gather(table, idx, ch):
    return _make_sc_row_gather(idx.shape[0], table.shape[1], ch)(table, idx)


def _gmm_kernel(be_ref, xs_ref, ew_ref, eb_ref, w_ref, out_ref):
    y = lax.dot_general(
        xs_ref[...], ew_ref[0], (((1,), (1,)), ((), ())),
        preferred_element_type=jnp.float32,
    )
    out_ref[...] = w_ref[...] * (y + eb_ref[0])


def _grouped_matmul(block_expert, x_sorted, w_pad, expert_W, expert_b):
    grid_spec = pltpu.PrefetchScalarGridSpec(
        num_scalar_prefetch=1,
        grid=(G,),
        in_specs=[
            pl.BlockSpec((BT, D_IN), lambda g, be: (g, 0)),
            pl.BlockSpec((1, D_OUT, D_IN), lambda g, be: (be[g], 0, 0)),
            pl.BlockSpec((1, 1, D_OUT), lambda g, be: (be[g], 0, 0)),
            pl.BlockSpec((BT, 1), lambda g, be: (g, 0)),
        ],
        out_specs=pl.BlockSpec((BT, D_OUT), lambda g, be: (g, 0)),
    )
    return pl.pallas_call(
        _gmm_kernel,
        grid_spec=grid_spec,
        out_shape=jax.ShapeDtypeStruct((P_PAD, D_OUT), jnp.float32),
        compiler_params=pltpu.CompilerParams(
            dimension_semantics=("arbitrary",),
        ),
    )(block_expert, x_sorted, expert_W, expert_b.reshape(E, 1, D_OUT),
      w_pad.reshape(P_PAD, 1))


def _add_kernel(a_ref, b_ref, o_ref):
    o_ref[...] = a_ref[...] + b_ref[...]


def _combine(gathered):
    nb = N // BLOCK_N
    return pl.pallas_call(
        _add_kernel,
        grid=(nb,),
        in_specs=[
            pl.BlockSpec((BLOCK_N, D_OUT), lambda n: (n, 0)),
            pl.BlockSpec((BLOCK_N, D_OUT), lambda n, _nb=nb: (n + _nb, 0)),
        ],
        out_specs=pl.BlockSpec((BLOCK_N, D_OUT), lambda n: (n, 0)),
        out_shape=jax.ShapeDtypeStruct((N, D_OUT), jnp.float32),
    )(gathered, gathered)


@jax.jit
def kernel(x, gate_W, gate_b, expert_W, expert_b):
    w8 = _gate(x, gate_W, gate_b)

    nz = w8 > 0.0
    e0 = jnp.argmax(nz, axis=1).astype(jnp.int32)
    e1 = (E - 1) - jnp.argmax(nz[:, ::-1], axis=1).astype(jnp.int32)
    rows = jnp.arange(N, dtype=jnp.int32)
    wa = w8[rows, e0]
    wb = jnp.where(e1 == e0, 0.0, w8[rows, e1])
    ef = jnp.concatenate([e0, e1])
    tok = jnp.concatenate([rows, rows])
    wf = jnp.concatenate([wa, wb])

    oh = (ef[:, None] == jnp.arange(E, dtype=jnp.int32)[None, :]).astype(jnp.int32)
    csum = jnp.cumsum(oh, axis=0)
    rank = csum[jnp.arange(P), ef] - 1
    counts = csum[-1]
    padded = ((counts + BT - 1) // BT) * BT
    offs = jnp.concatenate(
        [jnp.zeros((1,), jnp.int32), jnp.cumsum(padded)[:-1].astype(jnp.int32)]
    )
    pos = offs[ef] + rank
    tok_pad = jnp.zeros((P_PAD,), jnp.int32).at[pos].set(tok)
    w_pad = jnp.zeros((P_PAD,), jnp.float32).at[pos].set(wf)
    block_expert = jnp.repeat(
        jnp.arange(E, dtype=jnp.int32), padded // BT, total_repeat_length=G
    )

    x_sorted = _sc_row_gather(x, tok_pad, 48)
    y_sorted = _grouped_matmul(block_expert, x_sorted, w_pad, expert_W, expert_b)
    gathered = _sc_row_gather(y_sorted, pos, 32)
    return _combine(gathered)


# speedup vs baseline: 1.0891x; 1.0891x over previous
import functools

import jax
import jax.numpy as jnp
from jax import lax
from jax.experimental import pallas as pl
from jax.experimental.pallas import tpu as pltpu
from jax.experimental.pallas import tpu_sc as plsc

N, D_IN, D_OUT, E = 4096, 2048, 2048, 8
P = 2 * N
BT = 256
P_PAD = P + E * BT
G = P_PAD // BT
NC, NS = 2, 16
NW = NC * NS
BLOCK_N = 1024


def _gate_kernel(x_ref, gw_ref, gb_ref, w8_ref):
    gs = lax.dot_general(
        x_ref[...], gw_ref[...], (((1,), (1,)), ((), ())),
        preferred_element_type=jnp.float32,
    ) + gb_ref[...]
    lane = lax.broadcasted_iota(jnp.int32, gs.shape, 1)
    m1 = jnp.max(gs, axis=1, keepdims=True)
    i1 = jnp.min(jnp.where(gs == m1, lane, E), axis=1, keepdims=True)
    masked = jnp.where(lane == i1, -jnp.inf, gs)
    m2 = jnp.max(masked, axis=1, keepdims=True)
    i2 = jnp.min(jnp.where(masked == m2, lane, E), axis=1, keepdims=True)
    w1 = 1.0 / (1.0 + jnp.exp(m2 - m1))
    w8_ref[...] = jnp.where(lane == i1, w1, jnp.where(lane == i2, 1.0 - w1, 0.0))


def _gate(x, gate_W, gate_b):
    return pl.pallas_call(
        _gate_kernel,
        grid=(N // BLOCK_N,),
        in_specs=[
            pl.BlockSpec((BLOCK_N, D_IN), lambda n: (n, 0)),
            pl.BlockSpec((E, D_IN), lambda n: (0, 0)),
            pl.BlockSpec((1, E), lambda n: (0, 0)),
        ],
        out_specs=pl.BlockSpec((BLOCK_N, E), lambda n: (n, 0)),
        out_shape=jax.ShapeDtypeStruct((N, E), jnp.float32),
    )(x, gate_W, gate_b.reshape(1, E))


@functools.lru_cache(maxsize=None)
def _make_sc_row_gather(n_rows, d, ch):
    per_w = n_rows // NW
    nch = per_w // ch
    mesh = plsc.VectorSubcoreMesh(core_axis_name="c", subcore_axis_name="s")

    @functools.partial(
        pl.kernel,
        mesh=mesh,
        out_type=jax.ShapeDtypeStruct((n_rows, d), jnp.float32),
        scratch_types=[
            pltpu.VMEM((per_w,), jnp.int32),
            pltpu.VMEM((ch, d), jnp.float32),
            pltpu.VMEM((ch, d), jnp.float32),
            pltpu.SemaphoreType.DMA,
            pltpu.SemaphoreType.DMA,
        ],
    )
    def gather(table_hbm, idx_hbm, out_hbm, idx_v, buf0, buf1, sem0, sem1):
        wid = lax.axis_index("s") * NC + lax.axis_index("c")
        base = wid * per_w
        pltpu.sync_copy(idx_hbm.at[pl.ds(base, per_w)], idx_v)
        bufs, sems, cps = [buf0, buf1], [sem0, sem1], [None, None]
        cps[0] = pltpu.async_copy(
            table_hbm.at[idx_v.at[pl.ds(0, ch)]], bufs[0], sems[0])
        for i in range(nch):
            if i + 1 < nch:
                cps[(i + 1) % 2] = pltpu.async_copy(
                    table_hbm.at[idx_v.at[pl.ds((i + 1) * ch, ch)]],
                    bufs[(i + 1) % 2], sems[(i + 1) % 2])
            cps[i % 2].wait()
            pltpu.sync_copy(bufs[i % 2], out_hbm.at[pl.ds(base + i * ch, ch)])

    return gather


def _sc_row_gather(table, idx, ch):
    return _make_sc_row_gather(idx.shape[0], table.shape[1], ch)(table, idx)


def _gmm_kernel(be_ref, xs_ref, ew_ref, eb_ref, w_ref, out_ref):
    y = lax.dot_general(
        xs_ref[...], ew_ref[0], (((1,), (1,)), ((), ())),
        preferred_element_type=jnp.float32,
    )
    out_ref[...] = w_ref[...] * (y + eb_ref[0])


def _grouped_matmul(block_expert, x_sorted, w_pad, expert_W, expert_b):
    grid_spec = pltpu.PrefetchScalarGridSpec(
        num_scalar_prefetch=1,
        grid=(G,),
        in_specs=[
            pl.BlockSpec((BT, D_IN), lambda g, be: (g, 0)),
            pl.BlockSpec((1, D_OUT, D_IN), lambda g, be: (be[g], 0, 0)),
            pl.BlockSpec((1, 1, D_OUT), lambda g, be: (be[g], 0, 0)),
            pl.BlockSpec((BT, 1), lambda g, be: (g, 0)),
        ],
        out_specs=pl.BlockSpec((BT, D_OUT), lambda g, be: (g, 0)),
    )
    return pl.pallas_call(
        _gmm_kernel,
        grid_spec=grid_spec,
        out_shape=jax.ShapeDtypeStruct((P_PAD, D_OUT), jnp.float32),
        compiler_params=pltpu.CompilerParams(
            dimension_semantics=("arbitrary",),
        ),
    )(block_expert, x_sorted, expert_W, expert_b.reshape(E, 1, D_OUT),
      w_pad.reshape(P_PAD, 1))


def _add_kernel(a_ref, b_ref, o_ref):
    o_ref[...] = a_ref[...] + b_ref[...]


def _combine(gathered):
    nb = N // BLOCK_N
    return pl.pallas_call(
        _add_kernel,
        grid=(nb,),
        in_specs=[
            pl.BlockSpec((BLOCK_N, D_OUT), lambda n: (n, 0)),
            pl.BlockSpec((BLOCK_N, D_OUT), lambda n, _nb=nb: (n + _nb, 0)),
        ],
        out_specs=pl.BlockSpec((BLOCK_N, D_OUT), lambda n: (n, 0)),
        out_shape=jax.ShapeDtypeStruct((N, D_OUT), jnp.float32),
    )(gathered, gathered)


@jax.jit
def kernel(x, gate_W, gate_b, expert_W, expert_b):
    w8 = _gate(x, gate_W, gate_b)

    nz = w8 > 0.0
    e0 = jnp.argmax(nz, axis=1).astype(jnp.int32)
    e1 = (E - 1) - jnp.argmax(nz[:, ::-1], axis=1).astype(jnp.int32)
    rows = jnp.arange(N, dtype=jnp.int32)
    wa = w8[rows, e0]
    wb = jnp.where(e1 == e0, 0.0, w8[rows, e1])
    ef = jnp.concatenate([e0, e1])
    tok = jnp.concatenate([rows, rows])
    wf = jnp.concatenate([wa, wb])

    oh = (ef[:, None] == jnp.arange(E, dtype=jnp.int32)[None, :]).astype(jnp.int32)
    csum = jnp.cumsum(oh, axis=0)
    rank = csum[jnp.arange(P), ef] - 1
    counts = csum[-1]
    padded = ((counts + BT - 1) // BT) * BT
    offs = jnp.concatenate(
        [jnp.zeros((1,), jnp.int32), jnp.cumsum(padded)[:-1].astype(jnp.int32)]
    )
    pos = offs[ef] + rank
    tok_pad = jnp.zeros((P_PAD,), jnp.int32).at[pos].set(tok)
    w_pad = jnp.zeros((P_PAD,), jnp.float32).at[pos].set(wf)
    block_expert = jnp.repeat(
        jnp.arange(E, dtype=jnp.int32), padded // BT, total_repeat_length=G
    )

    x_sorted = _sc_row_gather(x, tok_pad, 16)
    y_sorted = _grouped_matmul(block_expert, x_sorted, w_pad, expert_W, expert_b)
    gathered = _sc_row_gather(y_sorted, pos, 16)
    return _combine(gathered)
